# Initial kernel scaffold; baseline (speedup 1.0000x reference)
import functools

import jax
import jax.numpy as jnp
from jax import lax
from jax.experimental import pallas as pl
from jax.experimental.pallas import tpu as pltpu

_DELTA_V = 0.5
_DELTA_D = 1.5
_NL = 20
_NS = 8
_SEG = _NL * _NS


def _loss_body(lab_ref, sb_ref, x_ref, out_ref,
               sums_ref, cnt_ref, pull_ref, coef_ref, meta_ref, *, nb, r):
    phase = pl.program_id(0)
    i = pl.program_id(1)

    @pl.when((phase == 0) & (i == 0))
    def _init():
        sums_ref[...] = jnp.zeros_like(sums_ref)
        cnt_ref[...] = jnp.zeros_like(cnt_ref)
        pull_ref[...] = jnp.zeros_like(pull_ref)

    lab = lab_ref[0]
    sb = sb_ref[0]
    seg = sb * _NL + lab
    seg_b = jnp.broadcast_to(seg, (_SEG, r))
    sid = lax.broadcasted_iota(jnp.int32, (_SEG, r), 0)
    oh = (seg_b == sid).astype(jnp.float32)

    x = x_ref[...]
    ssq = jnp.sum(x * x, axis=1, keepdims=True)
    scale = 1.0 / (jnp.sqrt(ssq) + 1e-8)
    xn = x * scale

    @pl.when(phase == 0)
    def _accumulate_sums():
        sums_ref[...] += lax.dot_general(
            oh, xn, (((1,), (0,)), ((), ())),
            preferred_element_type=jnp.float32)
        cnt_ref[...] += jnp.sum(oh, axis=1, keepdims=True)

    @pl.when((phase == 1) & (i == 0))
    def _finalize():
        cnt = cnt_ref[...]
        cnt_safe = jnp.maximum(cnt, 1.0)
        mu = sums_ref[...] / cnt_safe
        sums_ref[...] = mu
        present = (cnt > 0.0).astype(jnp.float32)

        sbid = lax.broadcasted_iota(jnp.int32, (_NS, _SEG), 0)
        segid2 = lax.broadcasted_iota(jnp.int32, (_NS, _SEG), 1)
        sb_oh = (segid2 // _NL == sbid).astype(jnp.float32)
        m_sb = lax.dot_general(sb_oh, present, (((1,), (0,)), ((), ())),
                               preferred_element_type=jnp.float32)
        m_safe = jnp.maximum(m_sb, 1.0)
        m_per_seg = lax.dot_general(sb_oh, m_safe, (((0,), (0,)), ((), ())),
                                    preferred_element_type=jnp.float32)
        coef_ref[...] = present / (m_per_seg * cnt_safe)

        pts_sb = lax.dot_general(sb_oh, cnt, (((1,), (0,)), ((), ())),
                                 preferred_element_type=jnp.float32)
        bval = jnp.sum((pts_sb > 0.0).astype(jnp.float32))

        segiota = lax.broadcasted_iota(jnp.float32, (_SEG, 1), 0)
        mu_push = mu + (1.0 - present) * (1.0e6 + 1.0e4 * segiota)
        push_total = jnp.float32(0.0)
        eye = (lax.broadcasted_iota(jnp.int32, (_NL, _NL), 0)
               == lax.broadcasted_iota(jnp.int32, (_NL, _NL), 1))
        for s in range(_NS):
            mus = mu_push[s * _NL:(s + 1) * _NL, :]
            p_col = present[s * _NL:(s + 1) * _NL, :]
            pd = jnp.sum(jnp.abs(mus[:, None, :] - mus[None, :, :]), axis=2)
            dists = jnp.maximum(2.0 * _DELTA_D - pd, 0.0)
            dm = jnp.where(eye, 0.0, dists)
            ms = jnp.sum(p_col)
            denom = jnp.where(ms > 1.0, ms * (ms - 1.0), 1.0)
            push_total = push_total + jnp.sum(dm * dm) / denom
        meta_ref[0] = push_total
        meta_ref[1] = bval

    @pl.when(phase == 1)
    def _accumulate_pull():
        mu = sums_ref[...]
        musel = lax.dot_general(oh, mu, (((0,), (0,)), ((), ())),
                                preferred_element_type=jnp.float32)
        d = jnp.sum(jnp.abs(musel - xn), axis=1, keepdims=True)
        t = jnp.maximum(d - _DELTA_V, 0.0)
        term = t * t
        pull_ref[...] += lax.dot_general(
            oh, term, (((1,), (0,)), ((), ())),
            preferred_element_type=jnp.float32)

    @pl.when((phase == 1) & (i == nb - 1))
    def _final():
        lp = jnp.sum(pull_ref[...] * coef_ref[...])
        out_ref[0, 0] = (lp + meta_ref[0]) / meta_ref[1]


def kernel(outputs, labels, subbatch_indices):
    n, d = outputs.shape
    r = 2000
    nb = n // r
    assert n % r == 0

    lab3 = labels.reshape(nb, 1, r)
    sb3 = subbatch_indices.reshape(nb, 1, r)

    body = functools.partial(_loss_body, nb=nb, r=r)
    out = pl.pallas_call(
        body,
        grid=(2, nb),
        in_specs=[
            pl.BlockSpec((1, 1, r), lambda p, i: (i, 0, 0)),
            pl.BlockSpec((1, 1, r), lambda p, i: (i, 0, 0)),
            pl.BlockSpec((r, d), lambda p, i: (i, 0)),
        ],
        out_specs=pl.BlockSpec((1, 1), lambda p, i: (0, 0)),
        out_shape=jax.ShapeDtypeStruct((1, 1), jnp.float32),
        scratch_shapes=[
            pltpu.VMEM((_SEG, d), jnp.float32),
            pltpu.VMEM((_SEG, 1), jnp.float32),
            pltpu.VMEM((_SEG, 1), jnp.float32),
            pltpu.VMEM((_SEG, 1), jnp.float32),
            pltpu.SMEM((2,), jnp.float32),
        ],
    )(lab3, sb3, outputs)
    return out[0, 0]


# speedup vs baseline: 39.0129x; 39.0129x over previous
import functools

import jax
import jax.numpy as jnp
from jax import lax
from jax.experimental import pallas as pl
from jax.experimental.pallas import tpu as pltpu

_DELTA_V = 0.5
_DELTA_D = 1.5
_NL = 20
_NS = 8
_SEG = _NL * _NS


def _loss_body(lab_ref, sb_ref, x_ref, out_ref,
               sums_ref, cnt_ref, pull_ref, coef_ref, meta_ref, *, nb, r):
    phase = pl.program_id(0)
    i = pl.program_id(1)

    @pl.when((phase == 0) & (i == 0))
    def _init():
        sums_ref[...] = jnp.zeros_like(sums_ref)
        cnt_ref[...] = jnp.zeros_like(cnt_ref)
        pull_ref[...] = jnp.zeros_like(pull_ref)

    lab = lab_ref[0]
    sb = sb_ref[0]
    seg = sb * _NL + lab
    seg_b = jnp.broadcast_to(seg, (_SEG, r))
    sid = lax.broadcasted_iota(jnp.int32, (_SEG, r), 0)
    oh = (seg_b == sid).astype(jnp.float32)

    x = x_ref[...]
    ssq = jnp.sum(x * x, axis=1, keepdims=True)
    scale = 1.0 / (jnp.sqrt(ssq) + 1e-8)
    xn = x * scale

    @pl.when(phase == 0)
    def _accumulate_sums():
        sums_ref[...] += lax.dot_general(
            oh, xn, (((1,), (0,)), ((), ())),
            preferred_element_type=jnp.float32)
        cnt_ref[...] += jnp.sum(oh, axis=1, keepdims=True)

    @pl.when((phase == 1) & (i == 0))
    def _finalize():
        cnt = cnt_ref[...]
        cnt_safe = jnp.maximum(cnt, 1.0)
        mu = sums_ref[...] / cnt_safe
        sums_ref[...] = mu
        present = (cnt > 0.0).astype(jnp.float32)

        sbid = lax.broadcasted_iota(jnp.int32, (_NS, _SEG), 0)
        segid2 = lax.broadcasted_iota(jnp.int32, (_NS, _SEG), 1)
        sb_oh = (segid2 // _NL == sbid).astype(jnp.float32)
        m_sb = lax.dot_general(sb_oh, present, (((1,), (0,)), ((), ())),
                               preferred_element_type=jnp.float32)
        m_safe = jnp.maximum(m_sb, 1.0)
        m_per_seg = lax.dot_general(sb_oh, m_safe, (((0,), (0,)), ((), ())),
                                    preferred_element_type=jnp.float32)
        coef_ref[...] = present / (m_per_seg * cnt_safe)

        pts_sb = lax.dot_general(sb_oh, cnt, (((1,), (0,)), ((), ())),
                                 preferred_element_type=jnp.float32)
        bval = jnp.sum((pts_sb > 0.0).astype(jnp.float32))

        segiota = lax.broadcasted_iota(jnp.int32, (_SEG, 1), 0).astype(jnp.float32)
        mu_push = mu + (1.0 - present) * (1.0e6 + 1.0e4 * segiota)
        push_total = jnp.float32(0.0)
        eye = (lax.broadcasted_iota(jnp.int32, (_NL, _NL), 0)
               == lax.broadcasted_iota(jnp.int32, (_NL, _NL), 1))
        for s in range(_NS):
            mus = mu_push[s * _NL:(s + 1) * _NL, :]
            p_col = present[s * _NL:(s + 1) * _NL, :]
            pd = jnp.sum(jnp.abs(mus[:, None, :] - mus[None, :, :]), axis=2)
            dists = jnp.maximum(2.0 * _DELTA_D - pd, 0.0)
            dm = jnp.where(eye, 0.0, dists)
            ms = jnp.sum(p_col)
            denom = jnp.where(ms > 1.0, ms * (ms - 1.0), 1.0)
            push_total = push_total + jnp.sum(dm * dm) / denom
        meta_ref[0] = push_total
        meta_ref[1] = bval

    @pl.when(phase == 1)
    def _accumulate_pull():
        mu = sums_ref[...]
        musel = lax.dot_general(oh, mu, (((0,), (0,)), ((), ())),
                                preferred_element_type=jnp.float32)
        d = jnp.sum(jnp.abs(musel - xn), axis=1, keepdims=True)
        t = jnp.maximum(d - _DELTA_V, 0.0)
        term = t * t
        pull_ref[...] += lax.dot_general(
            oh, term, (((1,), (0,)), ((), ())),
            preferred_element_type=jnp.float32)

    @pl.when((phase == 1) & (i == nb - 1))
    def _final():
        lp = jnp.sum(pull_ref[...] * coef_ref[...])
        loss = (lp + meta_ref[0]) / meta_ref[1]
        out_ref[...] = jnp.broadcast_to(loss, (1, 1))


def kernel(outputs, labels, subbatch_indices):
    n, d = outputs.shape
    r = 2000
    nb = n // r
    assert n % r == 0

    lab3 = labels.reshape(nb, 1, r)
    sb3 = subbatch_indices.reshape(nb, 1, r)

    body = functools.partial(_loss_body, nb=nb, r=r)
    out = pl.pallas_call(
        body,
        grid=(2, nb),
        in_specs=[
            pl.BlockSpec((1, 1, r), lambda p, i: (i, 0, 0)),
            pl.BlockSpec((1, 1, r), lambda p, i: (i, 0, 0)),
            pl.BlockSpec((r, d), lambda p, i: (i, 0)),
        ],
        out_specs=pl.BlockSpec((1, 1), lambda p, i: (0, 0)),
        out_shape=jax.ShapeDtypeStruct((1, 1), jnp.float32),
        scratch_shapes=[
            pltpu.VMEM((_SEG, d), jnp.float32),
            pltpu.VMEM((_SEG, 1), jnp.float32),
            pltpu.VMEM((_SEG, 1), jnp.float32),
            pltpu.VMEM((_SEG, 1), jnp.float32),
            pltpu.SMEM((2,), jnp.float32),
        ],
    )(lab3, sb3, outputs)
    return out[0, 0]


# speedup vs baseline: 42.3007x; 1.0843x over previous
import functools

import jax
import jax.numpy as jnp
from jax import lax
from jax.experimental import pallas as pl
from jax.experimental.pallas import tpu as pltpu

_DELTA_V = 0.5
_DELTA_D = 1.5
_NL = 20
_NS = 8
_SEG = _NL * _NS


def _loss_body(lab_ref, sb_ref, x_ref, out_ref,
               sums_ref, cnt_ref, pull_ref, coef_ref, meta_ref, *, nb, r):
    phase = pl.program_id(0)
    i = pl.program_id(1)

    @pl.when((phase == 0) & (i == 0))
    def _init():
        sums_ref[...] = jnp.zeros_like(sums_ref)
        cnt_ref[...] = jnp.zeros_like(cnt_ref)
        pull_ref[...] = jnp.zeros_like(pull_ref)

    lab = lab_ref[0]
    sb = sb_ref[0]
    seg = sb * _NL + lab
    seg_b = jnp.broadcast_to(seg, (_SEG, r))
    sid = lax.broadcasted_iota(jnp.int32, (_SEG, r), 0)
    oh = (seg_b == sid).astype(jnp.float32)

    x = x_ref[...]
    ones_d = jnp.ones((x.shape[1], 1), jnp.float32)
    ssq = lax.dot_general(x * x, ones_d, (((1,), (0,)), ((), ())),
                          preferred_element_type=jnp.float32)
    scale = 1.0 / (jnp.sqrt(ssq) + 1e-8)
    xn = x * scale

    @pl.when(phase == 0)
    def _accumulate_sums():
        sums_ref[...] += lax.dot_general(
            oh, xn, (((1,), (0,)), ((), ())),
            preferred_element_type=jnp.float32)
        cnt_ref[...] += jnp.sum(oh, axis=1, keepdims=True)

    @pl.when((phase == 1) & (i == 0))
    def _finalize():
        cnt = cnt_ref[...]
        cnt_safe = jnp.maximum(cnt, 1.0)
        mu = sums_ref[...] / cnt_safe
        sums_ref[...] = mu
        present = (cnt > 0.0).astype(jnp.float32)

        sbid = lax.broadcasted_iota(jnp.int32, (_NS, _SEG), 0)
        segid2 = lax.broadcasted_iota(jnp.int32, (_NS, _SEG), 1)
        sb_oh = (segid2 // _NL == sbid).astype(jnp.float32)
        m_sb = lax.dot_general(sb_oh, present, (((1,), (0,)), ((), ())),
                               preferred_element_type=jnp.float32)
        m_safe = jnp.maximum(m_sb, 1.0)
        m_per_seg = lax.dot_general(sb_oh, m_safe, (((0,), (0,)), ((), ())),
                                    preferred_element_type=jnp.float32)
        coef_ref[...] = present / (m_per_seg * cnt_safe)

        pts_sb = lax.dot_general(sb_oh, cnt, (((1,), (0,)), ((), ())),
                                 preferred_element_type=jnp.float32)
        bval = jnp.sum((pts_sb > 0.0).astype(jnp.float32))

        segiota = lax.broadcasted_iota(jnp.int32, (_SEG, 1), 0).astype(jnp.float32)
        mu_push = mu + (1.0 - present) * (1.0e6 + 1.0e4 * segiota)
        push_total = jnp.float32(0.0)
        eye = (lax.broadcasted_iota(jnp.int32, (_NL, _NL), 0)
               == lax.broadcasted_iota(jnp.int32, (_NL, _NL), 1))
        for s in range(_NS):
            mus = mu_push[s * _NL:(s + 1) * _NL, :]
            p_col = present[s * _NL:(s + 1) * _NL, :]
            pd = jnp.sum(jnp.abs(mus[:, None, :] - mus[None, :, :]), axis=2)
            dists = jnp.maximum(2.0 * _DELTA_D - pd, 0.0)
            dm = jnp.where(eye, 0.0, dists)
            ms = jnp.sum(p_col)
            denom = jnp.where(ms > 1.0, ms * (ms - 1.0), 1.0)
            push_total = push_total + jnp.sum(dm * dm) / denom
        meta_ref[0] = push_total
        meta_ref[1] = bval

    @pl.when(phase == 1)
    def _accumulate_pull():
        mu = sums_ref[...]
        musel = lax.dot_general(oh, mu, (((0,), (0,)), ((), ())),
                                preferred_element_type=jnp.float32)
        d = lax.dot_general(jnp.abs(musel - xn), ones_d,
                            (((1,), (0,)), ((), ())),
                            preferred_element_type=jnp.float32)
        t = jnp.maximum(d - _DELTA_V, 0.0)
        term = t * t
        pull_ref[...] += lax.dot_general(
            oh, term, (((1,), (0,)), ((), ())),
            preferred_element_type=jnp.float32)

    @pl.when((phase == 1) & (i == nb - 1))
    def _final():
        lp = jnp.sum(pull_ref[...] * coef_ref[...])
        loss = (lp + meta_ref[0]) / meta_ref[1]
        out_ref[...] = jnp.broadcast_to(loss, (1, 1))


def kernel(outputs, labels, subbatch_indices):
    n, d = outputs.shape
    r = 4000
    nb = n // r
    assert n % r == 0

    lab3 = labels.reshape(nb, 1, r)
    sb3 = subbatch_indices.reshape(nb, 1, r)

    body = functools.partial(_loss_body, nb=nb, r=r)
    out = pl.pallas_call(
        body,
        grid=(2, nb),
        in_specs=[
            pl.BlockSpec((1, 1, r), lambda p, i: (i, 0, 0)),
            pl.BlockSpec((1, 1, r), lambda p, i: (i, 0, 0)),
            pl.BlockSpec((r, d), lambda p, i: (i, 0)),
        ],
        out_specs=pl.BlockSpec((1, 1), lambda p, i: (0, 0)),
        out_shape=jax.ShapeDtypeStruct((1, 1), jnp.float32),
        scratch_shapes=[
            pltpu.VMEM((_SEG, d), jnp.float32),
            pltpu.VMEM((_SEG, 1), jnp.float32),
            pltpu.VMEM((_SEG, 1), jnp.float32),
            pltpu.VMEM((_SEG, 1), jnp.float32),
            pltpu.SMEM((2,), jnp.float32),
        ],
    )(lab3, sb3, outputs)
    return out[0, 0]
